# Initial kernel scaffold; baseline (speedup 1.0000x reference)
import jax
import jax.numpy as jnp
from jax.experimental import pallas as pl


def kernel(hidden_states, cos, sin, ln1_w, ln2_w, wqkv, wo, gate_w, w_gate, w_up, w_down):
    raise NotImplementedError("write your pallas kernel here")


# speedup vs baseline: 1.2618x; 1.2618x over previous
import functools

import jax
import jax.numpy as jnp
from jax.experimental import pallas as pl

B, S, D = 1, 2048, 1024
H, KVH, HD = 16, 8, 64
E, FF = 8, 2048
EPS = 1e-05
JITTER = 0.01
NEG = -1e30

TBLK = 256
NT = S // TBLK
FBLK = 512
NF = FF // FBLK


def _rmsnorm(x, w):
    return x * jax.lax.rsqrt(jnp.mean(x * x, axis=-1, keepdims=True) + EPS) * w


def _rope(x, cos, sin):
    rot = jnp.concatenate([-x[:, HD // 2:], x[:, :HD // 2]], axis=1)
    return x * cos + rot * sin


def _k1_body(x_ref, w_ref, wqkv_ref, out_ref):
    h = _rmsnorm(x_ref[...], w_ref[...])
    out_ref[...] = jnp.dot(h, wqkv_ref[...], preferred_element_type=jnp.float32)


def _qkv(x, ln1_w, wqkv):
    return pl.pallas_call(
        _k1_body,
        grid=(NT,),
        in_specs=[
            pl.BlockSpec((TBLK, D), lambda i: (i, 0)),
            pl.BlockSpec((1, D), lambda i: (0, 0)),
            pl.BlockSpec((D, (H + 2 * KVH) * HD), lambda i: (0, 0)),
        ],
        out_specs=pl.BlockSpec((TBLK, (H + 2 * KVH) * HD), lambda i: (i, 0)),
        out_shape=jax.ShapeDtypeStruct((S, (H + 2 * KVH) * HD), jnp.float32),
    )(x, ln1_w.reshape(1, D), wqkv)


def _k2_body(q_ref, k_ref, v_ref, cosq_ref, sinq_ref, cos_ref, sin_ref, o_ref):
    i = pl.program_id(1)
    q = _rope(q_ref[0], cosq_ref[...], sinq_ref[...]) * (1.0 / (HD ** 0.5))
    k = _rope(k_ref[0], cos_ref[...], sin_ref[...])
    logits = jax.lax.dot_general(
        q, k, (((1,), (1,)), ((), ())), preferred_element_type=jnp.float32)
    qi = i * TBLK + jax.lax.broadcasted_iota(jnp.int32, (TBLK, S), 0)
    kj = jax.lax.broadcasted_iota(jnp.int32, (TBLK, S), 1)
    logits = jnp.where(kj <= qi, logits, NEG)
    m = jnp.max(logits, axis=-1, keepdims=True)
    p = jnp.exp(logits - m)
    p = p / jnp.sum(p, axis=-1, keepdims=True)
    o_ref[0] = jnp.dot(p, v_ref[0], preferred_element_type=jnp.float32)


def _attention(q3, k3, v3, cos, sin):
    return pl.pallas_call(
        _k2_body,
        grid=(H, NT),
        in_specs=[
            pl.BlockSpec((1, TBLK, HD), lambda h, i: (h, i, 0)),
            pl.BlockSpec((1, S, HD), lambda h, i: (h // (H // KVH), 0, 0)),
            pl.BlockSpec((1, S, HD), lambda h, i: (h // (H // KVH), 0, 0)),
            pl.BlockSpec((TBLK, HD), lambda h, i: (i, 0)),
            pl.BlockSpec((TBLK, HD), lambda h, i: (i, 0)),
            pl.BlockSpec((S, HD), lambda h, i: (0, 0)),
            pl.BlockSpec((S, HD), lambda h, i: (0, 0)),
        ],
        out_specs=pl.BlockSpec((1, TBLK, HD), lambda h, i: (h, i, 0)),
        out_shape=jax.ShapeDtypeStruct((H, S, HD), jnp.float32),
    )(q3, k3, v3, cos, sin, cos, sin)


def _k3_body(o_ref, wo_ref, res_ref, w2_ref, gw_ref, r2_ref, xm_ref, lg_ref):
    r2 = jnp.dot(o_ref[...], wo_ref[...], preferred_element_type=jnp.float32) + res_ref[...]
    r2_ref[...] = r2
    h2 = _rmsnorm(r2, w2_ref[...])
    xm_ref[...] = h2
    lg_ref[...] = jnp.dot(h2, gw_ref[...], preferred_element_type=jnp.float32)


def _oproj_router(o2d, wo, resid, ln2_w, gate_w):
    return pl.pallas_call(
        _k3_body,
        grid=(NT,),
        in_specs=[
            pl.BlockSpec((TBLK, H * HD), lambda i: (i, 0)),
            pl.BlockSpec((H * HD, D), lambda i: (0, 0)),
            pl.BlockSpec((TBLK, D), lambda i: (i, 0)),
            pl.BlockSpec((1, D), lambda i: (0, 0)),
            pl.BlockSpec((D, E), lambda i: (0, 0)),
        ],
        out_specs=[
            pl.BlockSpec((TBLK, D), lambda i: (i, 0)),
            pl.BlockSpec((TBLK, D), lambda i: (i, 0)),
            pl.BlockSpec((TBLK, E), lambda i: (i, 0)),
        ],
        out_shape=[
            jax.ShapeDtypeStruct((S, D), jnp.float32),
            jax.ShapeDtypeStruct((S, D), jnp.float32),
            jax.ShapeDtypeStruct((S, E), jnp.float32),
        ],
    )(o2d, wo, resid, ln2_w.reshape(1, D), gate_w)


def _k4_body(sc_ref, comb_ref):
    scores = sc_ref[...]
    iota = jax.lax.broadcasted_iota(jnp.int32, (S, E), 1)
    mlt = jnp.max(scores, axis=-1, keepdims=True)
    idx1 = jnp.min(jnp.where(scores == mlt, iota, E), axis=-1, keepdims=True)
    oh1 = iota == idx1
    factor = jnp.maximum(jnp.abs(scores), mlt)
    mask = ((mlt - scores) / factor) > (2.0 * JITTER)
    mg = jnp.where(mask, NEG, scores)
    m = jnp.max(mg, axis=-1, keepdims=True)
    p = jnp.exp(mg - m)
    sm1 = p / jnp.sum(p, axis=-1, keepdims=True)
    mult1 = jnp.sum(jnp.where(oh1, sm1, 0.0), axis=-1, keepdims=True)

    msc = jnp.where(oh1, NEG, scores)
    mlt2 = jnp.max(msc, axis=-1, keepdims=True)
    idx2 = jnp.min(jnp.where(msc == mlt2, iota, E), axis=-1, keepdims=True)
    oh2 = iota == idx2
    factor2 = jnp.maximum(jnp.abs(scores), mlt2)
    mask2 = ((mlt2 - scores) / factor2) > (2.0 * JITTER)
    mg2 = jnp.where(mask2, NEG, msc)
    m2 = jnp.max(mg2, axis=-1, keepdims=True)
    p2 = jnp.exp(mg2 - m2)
    sm2 = p2 / jnp.sum(p2, axis=-1, keepdims=True)
    mult2 = jnp.sum(jnp.where(oh2, sm2, 0.0), axis=-1, keepdims=True)

    comb_ref[...] = jnp.where(oh1, mult1, 0.0) + jnp.where(oh2, mult2, 0.0)


def _router(logits):
    return pl.pallas_call(
        _k4_body,
        grid=(1,),
        in_specs=[pl.BlockSpec((S, E), lambda i: (0, 0))],
        out_specs=pl.BlockSpec((S, E), lambda i: (0, 0)),
        out_shape=jax.ShapeDtypeStruct((S, E), jnp.float32),
    )(logits)


def _k5_body(x_ref, wg_ref, wu_ref, wd_ref, c_ref, out_ref):
    e = pl.program_id(0)
    f = pl.program_id(1)

    @pl.when(jnp.logical_and(e == 0, f == 0))
    def _():
        out_ref[...] = jnp.zeros_like(out_ref)

    x = x_ref[...]
    a = jnp.dot(x, wg_ref[0], preferred_element_type=jnp.float32)
    g = (a * jax.nn.sigmoid(a)) * jnp.dot(x, wu_ref[0], preferred_element_type=jnp.float32)
    y = jnp.dot(g, wd_ref[0], preferred_element_type=jnp.float32)
    out_ref[...] += c_ref[0] * y


def _moe(xm, w_gate, w_up, w_down, combine_t):
    return pl.pallas_call(
        _k5_body,
        grid=(E, NF),
        in_specs=[
            pl.BlockSpec((S, D), lambda e, f: (0, 0)),
            pl.BlockSpec((1, D, FBLK), lambda e, f: (e, 0, f)),
            pl.BlockSpec((1, D, FBLK), lambda e, f: (e, 0, f)),
            pl.BlockSpec((1, FBLK, D), lambda e, f: (e, f, 0)),
            pl.BlockSpec((1, S, 1), lambda e, f: (e, 0, 0)),
        ],
        out_specs=pl.BlockSpec((S, D), lambda e, f: (0, 0)),
        out_shape=jax.ShapeDtypeStruct((S, D), jnp.float32),
    )(xm, w_gate, w_up, w_down, combine_t)


def kernel(hidden_states, cos, sin, ln1_w, ln2_w, wqkv, wo, gate_w, w_gate, w_up, w_down):
    x = hidden_states.reshape(S, D)
    qkv = _qkv(x, ln1_w, wqkv)
    q3 = qkv[:, : H * HD].reshape(S, H, HD).transpose(1, 0, 2)
    k3 = qkv[:, H * HD: (H + KVH) * HD].reshape(S, KVH, HD).transpose(1, 0, 2)
    v3 = qkv[:, (H + KVH) * HD:].reshape(S, KVH, HD).transpose(1, 0, 2)
    o3 = _attention(q3, k3, v3, cos, sin)
    o2d = o3.transpose(1, 0, 2).reshape(S, H * HD)
    residual2, xm, logits = _oproj_router(o2d, wo, x, ln2_w, gate_w)
    combine = _router(logits)
    out = _moe(xm, w_gate, w_up, w_down, combine.T.reshape(E, S, 1))
    return out.reshape(B, S, D), residual2.reshape(B, S, D)
